# baseline (speedup 1.0000x reference)
import functools

import jax
import jax.numpy as jnp
import numpy as np
from jax import lax
from jax.experimental import pallas as pl
from jax.experimental.pallas import tpu as pltpu
from jax.experimental.pallas import tpu_sc as plsc

D = 128
H = 8
B = 16384
RES = 2048
K = 80

ROWS = 1024
NW = 32
RPW = B // NW
CH = 64


def _l2norm_cols(x):
    n = jnp.sqrt(jnp.sum(x * x, axis=0, keepdims=True))
    return x / jnp.maximum(n, 1e-12)


def _tc1_body(sg_ref, sl_ref, vf_ref, w1g_ref, w1l_ref, w2g_ref, w2l_ref,
              wv_ref, wo_ref, b1g_ref, b1l_ref, b2g_ref, b2l_ref,
              bv_ref, bo_ref, fused_ref, fff_ref, f2_ref):
    f32 = jnp.float32

    def dot(w, x):
        return jnp.dot(w, x, preferred_element_type=f32)

    def dot_rt(w, x):
        return jax.lax.dot_general(w, x, (((1,), (1,)), ((), ())),
                                   preferred_element_type=f32)

    h = jax.nn.relu(dot_rt(w1g_ref[...], sg_ref[...]) + b1g_ref[...])
    h = dot(w2g_ref[...], h) + b2g_ref[...]
    sg2T = jax.nn.sigmoid(h[:D]) * h[D:]

    h = jax.nn.relu(dot_rt(w1l_ref[...], sl_ref[...]) + b1l_ref[...])
    h = dot(w2l_ref[...], h) + b2l_ref[...]
    sl2T = jax.nn.sigmoid(h[:D]) * h[D:]

    zT = jnp.concatenate([sg2T, sl2T], axis=1)
    zT = dot(wo_ref[...], dot(wv_ref[...], zT) + bv_ref[...]) + bo_ref[...]
    f1T = _l2norm_cols(zT[:, :ROWS])
    f2T = _l2norm_cols(zT[:, ROWS:])
    fffT = jax.nn.sigmoid(f1T) * f2T
    fused_ref[:, :RES] = vf_ref[...]
    fused_ref[:, RES:] = jnp.zeros((ROWS, D), f32)
    fff_ref[...] = fffT.T
    f2_ref[...] = f2T.T


def _patch_body(sp_ref, fused_in_any, fused_strip_ref):
    del fused_in_any
    fused_strip_ref[...] = sp_ref[...]


def _rev(x):
    return lax.rev(x, (0,))


def _sort(x):
    return plsc.sort_key_val(x, x)[0]


def _mm(a, b):
    return jnp.minimum(a, b), jnp.maximum(a, b)


def _m2(a, b):
    lo, hi = _mm(a, _rev(b))
    return _sort(lo), _sort(hi)


def _bm2(x0, x1):
    lo, hi = _mm(x0, x1)
    return _sort(lo), _sort(hi)


def _m4(A, Bq):
    x0, y0 = _mm(A[0], _rev(Bq[1]))
    x1, y1 = _mm(A[1], _rev(Bq[0]))
    return list(_bm2(x0, x1)) + list(_bm2(y0, y1))


def _bm4(x):
    l0, h0 = _mm(x[0], x[2])
    l1, h1 = _mm(x[1], x[3])
    return list(_bm2(l0, l1)) + list(_bm2(h0, h1))


def _m8(A, Bq):
    rB = [_rev(Bq[3]), _rev(Bq[2]), _rev(Bq[1]), _rev(Bq[0])]
    X = [jnp.minimum(A[k], rB[k]) for k in range(4)]
    Y = [jnp.maximum(A[k], rB[k]) for k in range(4)]
    return _bm4(X) + _bm4(Y)


def _row_threshold(a):
    s = [_sort(x) for x in a]
    p01 = _m2(s[0], s[1])
    p23 = _m2(s[2], s[3])
    p45 = _m2(s[4], s[5])
    p67 = _m2(s[6], s[7])
    q0 = _m4(list(p01), list(p23))
    q1 = _m4(list(p45), list(p67))
    o = _m8(q0, q1)
    return jnp.min(o[3])


def _make_sc_topk():
    mesh = plsc.VectorSubcoreMesh(core_axis_name="c", subcore_axis_name="s")

    @functools.partial(
        pl.kernel, mesh=mesh,
        out_type=jax.ShapeDtypeStruct((B, D), jnp.float32),
        scratch_types=[
            pltpu.VMEM((CH, D), jnp.float32),
            pltpu.VMEM((CH, D), jnp.float32),
        ],
        compiler_params=pltpu.CompilerParams(needs_layout_passes=False),
    )
    def sc_topk(fff_hbm, sparse_hbm, inbuf, outbuf):
        wid = lax.axis_index("s") * 2 + lax.axis_index("c")
        base = wid * RPW

        def do_row(j):
            v = [inbuf[j, pl.ds(k * 16, 16)] for k in range(8)]
            t = _row_threshold([jnp.abs(x) for x in v])
            for k in range(8):
                outbuf[j, pl.ds(k * 16, 16)] = jnp.where(
                    jnp.abs(v[k]) >= t, v[k], 0.0)

        def chunk_body(c, carry):
            r0 = base + c * CH
            pltpu.sync_copy(fff_hbm.at[pl.ds(r0, CH)], inbuf)

            def row_body(jj, carry2):
                do_row(2 * jj)
                do_row(2 * jj + 1)
                return carry2

            lax.fori_loop(0, CH // 2, row_body, 0)
            pltpu.sync_copy(outbuf, sparse_hbm.at[pl.ds(r0, CH)])
            return carry

        lax.fori_loop(0, RPW // CH, chunk_body, 0)

    return sc_topk


def _run_tc1(sg, sl, vf, w1g, w1l, w2g, w2l, wv, wo,
             b1g, b1l, b2g, b2l, bv, bo, *, interpret=False):
    grid = (B // ROWS,)
    row_spec = lambda c: pl.BlockSpec((ROWS, c), lambda i: (i, 0))
    full2 = lambda a, b: pl.BlockSpec((a, b), lambda i: (0, 0))
    return pl.pallas_call(
        _tc1_body,
        grid=grid,
        in_specs=[
            row_spec(D), row_spec(D), row_spec(RES),
            full2(2 * D, D), full2(2 * D, D),
            full2(2 * D, 2 * D), full2(2 * D, 2 * D),
            full2(D, D), full2(D, D),
            full2(2 * D, 1), full2(2 * D, 1),
            full2(2 * D, 1), full2(2 * D, 1),
            full2(D, 1), full2(D, 1),
        ],
        out_specs=[row_spec(RES + D), row_spec(D), row_spec(D)],
        out_shape=[
            jax.ShapeDtypeStruct((B, RES + D), jnp.float32),
            jax.ShapeDtypeStruct((B, D), jnp.float32),
            jax.ShapeDtypeStruct((B, D), jnp.float32),
        ],
        compiler_params=pltpu.CompilerParams(
            dimension_semantics=("arbitrary",),
        ),
        interpret=interpret,
    )(sg, sl, vf, w1g, w1l, w2g, w2l, wv, wo, b1g, b1l, b2g, b2l, bv, bo)


def _run_patch(sp, fused0, *, interpret=False):
    grid = (B // ROWS,)
    return pl.pallas_call(
        _patch_body,
        grid=grid,
        in_specs=[
            pl.BlockSpec((ROWS, D), lambda i: (i, 0)),
            pl.BlockSpec(memory_space=pl.ANY),
        ],
        out_specs=pl.BlockSpec((ROWS, D), lambda i: (i, RES // D)),
        out_shape=jax.ShapeDtypeStruct((B, RES + D), jnp.float32),
        input_output_aliases={1: 0},
        compiler_params=pltpu.CompilerParams(
            dimension_semantics=("arbitrary",),
        ),
        interpret=interpret,
    )(sp, fused0)


def kernel(semantic_global, semantic_local, visual_feat, params):
    p = params
    f32 = jnp.float32

    def blockdiag(a, b):
        z = jnp.zeros((D, D), f32)
        return jnp.block([[a, z], [z, b]])

    w1g = jnp.concatenate([p['W_gu1'], p['W_gd1']], axis=0)
    w1l = jnp.concatenate([p['W_lu1'], p['W_ld1']], axis=0)
    w2g = blockdiag(p['W_gu2'], p['W_gd2'])
    w2l = blockdiag(p['W_lu2'], p['W_ld2'])
    wv = p['W_v']
    wo = p['W_o']
    b1g = jnp.concatenate([p['b_gu1'], p['b_gd1']])[:, None]
    b1l = jnp.concatenate([p['b_lu1'], p['b_ld1']])[:, None]
    b2g = jnp.concatenate([p['b_gu2'], p['b_gd2']])[:, None]
    b2l = jnp.concatenate([p['b_lu2'], p['b_ld2']])[:, None]
    bv = p['b_v'][:, None]
    bo = p['b_o'][:, None]
    fused0, fff, f2 = _run_tc1(semantic_global, semantic_local, visual_feat,
                               w1g, w1l, w2g, w2l, wv, wo,
                               b1g, b1l, b2g, b2l, bv, bo)
    sparse = _make_sc_topk()(fff)
    fused = _run_patch(sparse, fused0)
    return fused, fff, f2
